# baseline (speedup 1.0000x reference)
import functools

import jax
import jax.numpy as jnp
from jax import lax
from jax.experimental import pallas as pl
from jax.experimental.pallas import tpu as pltpu
from jax.experimental.pallas import tpu_sc as plsc

B, T, H = 1024, 200, 64
V = 100000
BT = B * T
NC, NS = 2, 16
NW = NC * NS
IDXCH = 4096
NIC = BT // IDXCH
KP = 12
OP = 8
SP = 6

_MESH = plsc.VectorSubcoreMesh(core_axis_name="c", subcore_axis_name="s",
                               num_cores=NC, num_subcores=NS)


@functools.partial(
    pl.kernel,
    out_type=(
        jax.ShapeDtypeStruct((T, 7, 8, 8, 8, 128), jnp.float32),
        jax.ShapeDtypeStruct((T, 8, 8, 8, 8, 128), jnp.float32),
        jax.ShapeDtypeStruct((7, 8, 8, 8, 128), jnp.float32),
    ),
    mesh=_MESH,
    compiler_params=pltpu.CompilerParams(use_tc_tiling_on_sc=False,
                                         needs_layout_passes=False),
    scratch_types=[
        pltpu.VMEM((V,), jnp.float32),
        pltpu.VMEM((IDXCH,), jnp.int32),
        pltpu.VMEM((IDXCH // 128, 128), jnp.float32),
        pltpu.SemaphoreType.DMA,
    ],
)
def _sc_hgather(kidxf, oidxf, sidxf, ktabt, otabt, stabt,
                tk_out, tob_out, s_out,
                row_v, idx_v, out_v, sem0):
    wid = lax.axis_index("s") * NC + lax.axis_index("c")

    def gather_chunk(n16):
        def inner(l, c):
            for u in range(8):
                vec = plsc.load_gather(
                    row_v, [idx_v[pl.ds(l * 128 + u * 16, 16)]])
                out_v[(l * 8 + u) // 8, pl.ds(((l * 8 + u) % 8) * 16, 16)] \
                    = vec
            return c
        lax.fori_loop(0, n16 // 8, inner, 0)

    def do_unit(tabt, idxf, out, f, h):
        pltpu.async_copy(tabt.at[f, h], row_v, sem0).wait()
        ht = h // 8
        hi = h - ht * 8

        def chunk(c, cc):
            pltpu.sync_copy(idxf.at[f, pl.ds(c * IDXCH, IDXCH)], idx_v)
            gather_chunk(IDXCH // 16)
            for q in range(IDXCH // B):
                pltpu.sync_copy(out_v.at[pl.ds(q * 8, 8), :],
                                out.at[c * (IDXCH // B) + q, f, ht, :, hi, :])
            return cc
        lax.fori_loop(0, NIC, chunk, 0)

    @pl.when(wid < 16)
    def _k_units():
        for jj in range(KP):
            p = wid * KP + jj
            do_unit(ktabt, kidxf, tk_out, p // H, p % H)

    @pl.when(wid >= 16)
    def _o_units():
        for jj in range(OP):
            p = (wid - 16) * OP + jj
            do_unit(otabt, oidxf, tob_out, p // H, p % H)

    for jj in range(SP):
        p = wid * SP + jj
        f = p // H
        h = p - f * H
        pltpu.async_copy(stabt.at[f, h], row_v, sem0).wait()
        ht = h // 8
        hi = h - ht * 8
        pltpu.sync_copy(sidxf.at[f], idx_v.at[pl.ds(0, B)])
        gather_chunk(B // 16)
        pltpu.sync_copy(out_v.at[pl.ds(0, 8), :], s_out.at[f, ht, :, hi, :])


def _cont_packed_body(prev_ref, vals_ref, emb_ref, bias_ref, out_ref):
    out_ref[0, 0] = (emb_ref[0][:, None, :, None]
                     * vals_ref[0, 0][None, :, None, :]
                     + bias_ref[0][:, None, :, None])


def _cont_packed(buf, vals_p, emb_p, bias_p, ncat):
    nf = emb_p.shape[0]
    return pl.pallas_call(
        _cont_packed_body,
        grid=(T, nf),
        in_specs=[
            pl.BlockSpec(memory_space=pl.ANY),
            pl.BlockSpec((1, 1, 8, 128), lambda i, j: (j, i, 0, 0)),
            pl.BlockSpec((1, 8, 8), lambda i, j: (j, 0, 0)),
            pl.BlockSpec((1, 8, 8), lambda i, j: (j, 0, 0)),
        ],
        out_specs=pl.BlockSpec(
            (1, 1, 8, 8, 8, 128),
            lambda i, j, _n=ncat: (i, _n + j, 0, 0, 0, 0)),
        out_shape=jax.ShapeDtypeStruct(buf.shape, jnp.float32),
        input_output_aliases={0: 0},
    )(buf, vals_p, emb_p, bias_p)


def _sinp_packed_body(prev_ref, vals_ref, emb_ref, bias_ref, out_ref):
    out_ref[0] = (emb_ref[0][:, None, :, None]
                  * vals_ref[0][None, :, None, :]
                  + bias_ref[0][:, None, :, None])


def _sinp_packed(buf, vals_p, emb_p, bias_p):
    return pl.pallas_call(
        _sinp_packed_body,
        grid=(4,),
        in_specs=[
            pl.BlockSpec(memory_space=pl.ANY),
            pl.BlockSpec((1, 8, 128), lambda j: (j, 0, 0)),
            pl.BlockSpec((1, 8, 8), lambda j: (j, 0, 0)),
            pl.BlockSpec((1, 8, 8), lambda j: (j, 0, 0)),
        ],
        out_specs=pl.BlockSpec((1, 8, 8, 8, 128),
                               lambda j: (3 + j, 0, 0, 0, 0)),
        out_shape=jax.ShapeDtypeStruct(buf.shape, jnp.float32),
        input_output_aliases={0: 0},
    )(buf, vals_p, emb_p, bias_p)


def _tgt_body(vals_ref, emb_ref, bias_ref, out_ref):
    out_ref[0, 0] = (emb_ref[0][:, None] * vals_ref[0, 0, 0][None, :]
                     + bias_ref[0][:, None])


def _tgt_fill(vals_t, emb, bias):
    return pl.pallas_call(
        _tgt_body,
        grid=(T,),
        in_specs=[
            pl.BlockSpec((1, 1, 1, B), lambda i: (0, i, 0, 0)),
            pl.BlockSpec((1, H), lambda i: (0, 0)),
            pl.BlockSpec((1, H), lambda i: (0, 0)),
        ],
        out_specs=pl.BlockSpec((1, 1, H, B), lambda i: (i, 0, 0, 0)),
        out_shape=jax.ShapeDtypeStruct((T, 1, H, B), jnp.float32),
    )(vals_t.reshape(1, T, 1, B), emb, bias)


def _unpack(x):
    nd = x.ndim
    perm = tuple(range(nd - 4)) + (nd - 4, nd - 2, nd - 3, nd - 1)
    y = x.transpose(perm)
    y = y.reshape(x.shape[:-4] + (H, B))
    return jnp.moveaxis(y, -1, 0)


def kernel(s_cat, s_cont, k_cat, k_cont, o_cat, o_cont, target,
           s_cat_tables, k_cat_tables, o_cat_tables,
           s_cont_emb, s_cont_bias, k_cont_emb, k_cont_bias,
           o_cont_emb, o_cont_bias, tgt_emb, tgt_bias):
    kidxf = k_cat.transpose(2, 1, 0).reshape(3, BT)
    oidxf = o_cat.transpose(2, 1, 0).reshape(2, BT)
    sidxf = s_cat[:, 0, :].T
    ktabt = k_cat_tables.transpose(0, 2, 1)
    otabt = o_cat_tables.transpose(0, 2, 1)
    stabt = s_cat_tables.transpose(0, 2, 1)

    tkp0, tobp0, sp0 = _sc_hgather(kidxf, oidxf, sidxf, ktabt, otabt, stabt)

    kvals = k_cont.transpose(2, 1, 0).reshape(4, T, 8, 128)
    ovals = o_cont.transpose(2, 1, 0).reshape(6, T, 8, 128)
    svals = s_cont[:, 0, :].T.reshape(4, 8, 128)
    tvals = target.transpose(2, 1, 0)

    tkp = _cont_packed(tkp0, kvals, k_cont_emb.reshape(4, 8, 8),
                       k_cont_bias.reshape(4, 8, 8), 3)
    tobp = _cont_packed(tobp0, ovals, o_cont_emb.reshape(6, 8, 8),
                        o_cont_bias.reshape(6, 8, 8), 2)
    sp = _sinp_packed(sp0, svals, s_cont_emb.reshape(4, 8, 8),
                      s_cont_bias.reshape(4, 8, 8))
    ttgt = _tgt_fill(tvals, tgt_emb, tgt_bias)

    return (_unpack(sp),
            _unpack(tkp),
            _unpack(tobp),
            ttgt.transpose(3, 0, 1, 2))


# speedup vs baseline: 2.4484x; 2.4484x over previous
import functools

import jax
import jax.numpy as jnp
from jax import lax
from jax.experimental import pallas as pl
from jax.experimental.pallas import tpu as pltpu
from jax.experimental.pallas import tpu_sc as plsc

B, T, H = 1024, 200, 64
V = 100000
BT = B * T
NC, NS = 2, 16
NW = NC * NS
RPW = BT // NW
CH = 128
NCHUNK = RPW // CH
SB = B // NW
HB = 2 * H

_MESH = plsc.VectorSubcoreMesh(core_axis_name="c", subcore_axis_name="s",
                               num_cores=NC, num_subcores=NS)


def _pack_dst(out, i, p0, n):
    t_ = p0 // B
    rem = p0 - t_ * B
    half = rem // (B // 2)
    u = rem - half * (B // 2)
    return out.at[i, pl.ds(t_ * (B // 2) + u, n), pl.ds(half * H, H)]


def _run_field(tab, idx_slice, out, i, row0, idx_v, rv0, rv1, sem0, sem1):
    pltpu.sync_copy(idx_slice, idx_v)

    def start(g, rv, sem):
        pltpu.async_copy(tab.at[idx_v.at[g]], rv, sem)

    def drain(rv, sem):
        pltpu.make_async_copy(tab.at[pl.ds(0, CH)], rv, sem).wait()

    def store(g, rv):
        pltpu.sync_copy(rv, _pack_dst(out, i, row0 + g * CH, CH))

    start(0, rv0, sem0)

    def pair(it, c):
        g0 = 2 * it
        start(g0 + 1, rv1, sem1)
        drain(rv0, sem0)
        store(g0, rv0)
        start(g0 + 2, rv0, sem0)
        drain(rv1, sem1)
        store(g0 + 1, rv1)
        return c

    lax.fori_loop(0, NCHUNK // 2 - 1, pair, 0)
    g = NCHUNK - 2
    start(g + 1, rv1, sem1)
    drain(rv0, sem0)
    store(g, rv0)
    drain(rv1, sem1)
    store(g + 1, rv1)


_SC_SCRATCH = [
    pltpu.VMEM((NCHUNK, CH), jnp.int32),
    pltpu.VMEM((CH, H), jnp.float32),
    pltpu.VMEM((CH, H), jnp.float32),
    pltpu.SemaphoreType.DMA,
    pltpu.SemaphoreType.DMA,
]


@functools.partial(
    pl.kernel,
    out_type=jax.ShapeDtypeStruct((3, BT // 2, HB), jnp.float32),
    mesh=_MESH,
    compiler_params=pltpu.CompilerParams(use_tc_tiling_on_sc=False),
    scratch_types=_SC_SCRATCH,
)
def _sc_gather_k(kidx, ktab, kst_out, idx_v, rv0, rv1, sem0, sem1):
    wid = lax.axis_index("s") * NC + lax.axis_index("c")
    row0 = wid * RPW
    for i in range(3):
        _run_field(ktab, kidx.at[i, wid], kst_out, i, row0,
                   idx_v, rv0, rv1, sem0, sem1)


@functools.partial(
    pl.kernel,
    out_type=(
        jax.ShapeDtypeStruct((2, BT // 2, HB), jnp.float32),
        jax.ShapeDtypeStruct((3, B // 2, HB), jnp.float32),
    ),
    mesh=_MESH,
    compiler_params=pltpu.CompilerParams(use_tc_tiling_on_sc=False),
    scratch_types=_SC_SCRATCH + [
        pltpu.VMEM((SB,), jnp.int32),
        pltpu.VMEM((SB, H), jnp.float32),
    ],
)
def _sc_gather_os(oidx, sidx, otab, stab, ost_out, sst_out,
                  idx_v, rv0, rv1, sem0, sem1, sidx_v, srv):
    wid = lax.axis_index("s") * NC + lax.axis_index("c")
    row0 = wid * RPW

    for i in range(3):
        pltpu.sync_copy(sidx.at[i, wid], sidx_v)
        pltpu.async_copy(stab.at[sidx_v], srv, sem0).wait()
        pltpu.sync_copy(srv, _pack_dst(sst_out, i, wid * SB, SB))

    for i in range(2):
        _run_field(otab, oidx.at[i, wid], ost_out, i, row0,
                   idx_v, rv0, rv1, sem0, sem1)


def _unpack_t(s):
    return jnp.concatenate([s[:, :H].T, s[:, H:].T], axis=1)


def _assemble_body(ncat, stage_ref, vals_ref, emb_ref, bias_ref, out_ref):
    for i in range(ncat):
        out_ref[0, i] = _unpack_t(stage_ref[i, 0])
    nf = emb_ref.shape[0]
    for j in range(nf):
        out_ref[0, ncat + j] = (emb_ref[j][:, None]
                                * vals_ref[j, 0, 0][None, :]
                                + bias_ref[j][:, None])


def _assemble(stage, vals_t, emb, bias, ncat):
    nf = emb.shape[0]
    f_total = ncat + nf
    return pl.pallas_call(
        functools.partial(_assemble_body, ncat),
        grid=(T,),
        in_specs=[
            pl.BlockSpec((ncat, 1, B // 2, HB), lambda i: (0, i, 0, 0)),
            pl.BlockSpec((nf, 1, 1, B), lambda i: (0, i, 0, 0)),
            pl.BlockSpec((nf, H), lambda i: (0, 0)),
            pl.BlockSpec((nf, H), lambda i: (0, 0)),
        ],
        out_specs=pl.BlockSpec((1, f_total, H, B), lambda i: (i, 0, 0, 0)),
        out_shape=jax.ShapeDtypeStruct((T, f_total, H, B), jnp.float32),
    )(stage, vals_t.reshape(nf, T, 1, B), emb, bias)


def _sinp_body(stage_ref, vals_ref, emb_ref, bias_ref, out_ref):
    for i in range(3):
        out_ref[i] = _unpack_t(stage_ref[i])
    for j in range(4):
        out_ref[3 + j] = (emb_ref[j][:, None] * vals_ref[j][None, :]
                          + bias_ref[j][:, None])


def _sinp_assemble(stage, vals_t, emb, bias):
    return pl.pallas_call(
        _sinp_body,
        grid=(1,),
        in_specs=[
            pl.BlockSpec((3, B // 2, HB), lambda i: (0, 0, 0)),
            pl.BlockSpec((4, B), lambda i: (0, 0)),
            pl.BlockSpec((4, H), lambda i: (0, 0)),
            pl.BlockSpec((4, H), lambda i: (0, 0)),
        ],
        out_specs=pl.BlockSpec((7, H, B), lambda i: (0, 0, 0)),
        out_shape=jax.ShapeDtypeStruct((7, H, B), jnp.float32),
    )(stage, vals_t, emb, bias)


def _tgt_body(vals_ref, emb_ref, bias_ref, out_ref):
    out_ref[0, 0] = (emb_ref[0][:, None] * vals_ref[0, 0, 0][None, :]
                     + bias_ref[0][:, None])


def _tgt_fill(vals_t, emb, bias):
    return pl.pallas_call(
        _tgt_body,
        grid=(T,),
        in_specs=[
            pl.BlockSpec((1, 1, 1, B), lambda i: (0, i, 0, 0)),
            pl.BlockSpec((1, H), lambda i: (0, 0)),
            pl.BlockSpec((1, H), lambda i: (0, 0)),
        ],
        out_specs=pl.BlockSpec((1, 1, H, B), lambda i: (i, 0, 0, 0)),
        out_shape=jax.ShapeDtypeStruct((T, 1, H, B), jnp.float32),
    )(vals_t.reshape(1, T, 1, B), emb, bias)


def kernel(s_cat, s_cont, k_cat, k_cont, o_cat, o_cont, target,
           s_cat_tables, k_cat_tables, o_cat_tables,
           s_cont_emb, s_cont_bias, k_cont_emb, k_cont_bias,
           o_cont_emb, o_cont_bias, tgt_emb, tgt_bias):
    koff = jnp.arange(3, dtype=jnp.int32) * V
    ooff = jnp.arange(2, dtype=jnp.int32) * V
    kidx = (k_cat + koff).transpose(2, 1, 0).reshape(3, NW, NCHUNK, CH)
    oidx = (o_cat + ooff).transpose(2, 1, 0).reshape(2, NW, NCHUNK, CH)
    sidx = (s_cat[:, 0, :] + koff).T.reshape(3, NW, SB)
    ktab = k_cat_tables.reshape(3 * V, H)
    otab = o_cat_tables.reshape(2 * V, H)
    stab = s_cat_tables.reshape(3 * V, H)

    kst = _sc_gather_k(kidx, ktab)
    oidx, sidx = lax.optimization_barrier((oidx, sidx, kst))[:2]
    ost, sst = _sc_gather_os(oidx, sidx, otab, stab)

    kvals = k_cont.transpose(2, 1, 0)
    ovals = o_cont.transpose(2, 1, 0)
    svals = s_cont[:, 0, :].T
    tvals = target.transpose(2, 1, 0)

    tk = _assemble(kst.reshape(3, T, B // 2, HB), kvals,
                   k_cont_emb, k_cont_bias, 3)
    tob = _assemble(ost.reshape(2, T, B // 2, HB), ovals,
                    o_cont_emb, o_cont_bias, 2)
    sin = _sinp_assemble(sst, svals, s_cont_emb, s_cont_bias)
    ttgt = _tgt_fill(tvals, tgt_emb, tgt_bias)

    return (sin.transpose(2, 0, 1),
            tk.transpose(3, 0, 1, 2),
            tob.transpose(3, 0, 1, 2),
            ttgt.transpose(3, 0, 1, 2))
